# baseline (speedup 1.0000x reference)
import functools

import jax
import jax.numpy as jnp
from jax import lax
from jax.experimental import pallas as pl
from jax.experimental.pallas import tpu as pltpu
from jax.experimental.pallas import tpu_sc as plsc

R = 512
C = 32
NC = 2
NS = 16
NW = NC * NS
BC = 128
SCH = 8
SBC = BC * SCH
LANES = 16


def _prep_body(xyz_ref, idx_ref, w_ref):
    xyz_b = xyz_ref[...].astype(jnp.bfloat16).astype(jnp.float32)
    x = xyz_b[0:1, :]
    y = xyz_b[1:2, :]
    z = xyz_b[2:3, :]
    idx_rows = []
    w_rows = []
    for p, (gx, gy) in enumerate(((y, x), (z, x), (y, z))):
        ixf = (gx + 1.0) * (0.5 * (R - 1))
        iyf = (gy + 1.0) * (0.5 * (R - 1))
        ix0 = jnp.clip(jnp.floor(ixf), 0.0, R - 2.0)
        iy0 = jnp.clip(jnp.floor(iyf), 0.0, R - 2.0)
        fx = ixf - ix0
        fy = iyf - iy0
        base = (p * R * R) + iy0.astype(jnp.int32) * R + ix0.astype(jnp.int32)
        for dy in (0, 1):
            for dx in (0, 1):
                idx_rows.append(base + (dy * R + dx))
                w_rows.append((fx if dx else 1.0 - fx) * (fy if dy else 1.0 - fy))
    idx_ref[...] = jnp.concatenate(idx_rows, axis=0)
    w_ref[...] = jnp.concatenate(w_rows, axis=0)


@functools.lru_cache(maxsize=None)
def _make_prep(m):
    bp = 8192
    grid = (m // bp,)
    return pl.pallas_call(
        _prep_body,
        grid=grid,
        in_specs=[pl.BlockSpec((3, bp), lambda i: (0, i))],
        out_specs=[
            pl.BlockSpec((12, bp), lambda i: (0, i)),
            pl.BlockSpec((12, bp), lambda i: (0, i)),
        ],
        out_shape=[
            jax.ShapeDtypeStruct((12, m), jnp.int32),
            jax.ShapeDtypeStruct((12, m), jnp.float32),
        ],
    )


@functools.lru_cache(maxsize=None)
def _make_sc(m):
    ppt = m // NW
    nsch = ppt // SBC
    mesh = plsc.VectorSubcoreMesh(core_axis_name="c", subcore_axis_name="s")

    def compute_chunk(cc, rows, w_v, out_v, lanes):
        hi_mask = jnp.full((LANES,), -65536, jnp.int32)

        def group_body(g, carry2):
            rowbase = g * LANES
            row_lanes = lanes + rowbase
            wvs = [plsc.load_gather(w_v, [row_lanes + (j * SBC + cc * BC)])
                   for j in range(12)]
            out_lanes = row_lanes + cc * BC
            for w in range(C // 2):
                widx = jnp.full((LANES,), w, jnp.int32)
                acc_lo = jnp.zeros((LANES,), jnp.float32)
                acc_hi = jnp.zeros((LANES,), jnp.float32)
                for j in range(12):
                    word = plsc.load_gather(rows[j], [row_lanes, widx])
                    lo = plsc.bitcast(word << 16, jnp.float32)
                    hi = plsc.bitcast(word & hi_mask, jnp.float32)
                    acc_lo = acc_lo + wvs[j] * lo
                    acc_hi = acc_hi + wvs[j] * hi
                plsc.store_scatter(
                    out_v, [out_lanes, jnp.full((LANES,), 2 * w, jnp.int32)],
                    acc_lo)
                plsc.store_scatter(
                    out_v, [out_lanes, jnp.full((LANES,), 2 * w + 1, jnp.int32)],
                    acc_hi)
            return carry2

        lax.fori_loop(0, BC // LANES, group_body, 0)

    def body(table_hbm, idx_hbm, w_hbm, out_hbm,
             idx_v, w_v, rows_a, rows_b, out_v, sem_s, sem_a, sem_b):
        wid = lax.axis_index("s") * NC + lax.axis_index("c")
        lanes = lax.iota(jnp.int32, LANES)
        rows = (rows_a, rows_b)
        sems = (sem_a, sem_b)

        def fire_gathers(cc):
            par = cc % 2
            return [
                pltpu.async_copy(
                    table_hbm.at[idx_v.at[j, pl.ds(cc * BC, BC)]],
                    rows[par][j], sems[par])
                for j in range(12)
            ]

        def sch_body(s, carry):
            base = wid * ppt + s * SBC
            stages = []
            for j in range(12):
                stages.append(pltpu.async_copy(
                    idx_hbm.at[j, pl.ds(base, SBC)], idx_v.at[j], sem_s))
                stages.append(pltpu.async_copy(
                    w_hbm.at[j, pl.ds(base, SBC)],
                    w_v.at[pl.ds(j * SBC, SBC)], sem_s))
            for st in stages:
                st.wait()
            pending = fire_gathers(0)
            for cc in range(SCH):
                nxt = fire_gathers(cc + 1) if cc + 1 < SCH else []
                for g_ in pending:
                    g_.wait()
                compute_chunk(cc, rows[cc % 2], w_v, out_v, lanes)
                pending = nxt
            pltpu.sync_copy(out_v, out_hbm.at[pl.ds(base, SBC)])
            return carry

        lax.fori_loop(0, nsch, sch_body, 0)

    return pl.kernel(
        body,
        out_type=jax.ShapeDtypeStruct((m, C), jnp.float32),
        mesh=mesh,
        compiler_params=pltpu.CompilerParams(
            needs_layout_passes=False, use_tc_tiling_on_sc=False),
        scratch_types=[
            pltpu.VMEM((12, SBC), jnp.int32),
            pltpu.VMEM((12 * SBC,), jnp.float32),
            [pltpu.VMEM((BC, C // 2), jnp.int32) for _ in range(12)],
            [pltpu.VMEM((BC, C // 2), jnp.int32) for _ in range(12)],
            pltpu.VMEM((SBC, C), jnp.float32),
            pltpu.SemaphoreType.DMA,
            pltpu.SemaphoreType.DMA,
            pltpu.SemaphoreType.DMA,
        ],
    )


def kernel(xyz, oid, triplane):
    m = xyz.shape[0]
    if triplane.shape[0] == 1:
        planes = triplane.reshape(triplane.shape[1:])
    else:
        planes = triplane[oid]
    t32 = jnp.transpose(planes, (0, 2, 3, 1)).reshape(3 * R * R, C)
    bits = jax.lax.bitcast_convert_type(t32, jnp.int32)
    h = jax.lax.shift_right_logical(
        bits + 0x7FFF + (jax.lax.shift_right_logical(bits, 16) & 1), 16)
    table = (h[:, 0::2] & 0xFFFF) | (h[:, 1::2] << 16)
    xyz_t = xyz.T
    idx6, w12 = _make_prep(m)(xyz_t)
    return _make_sc(m)(table, idx6, w12)


# speedup vs baseline: 5.3361x; 5.3361x over previous
import functools

import jax
import jax.numpy as jnp
from jax import lax
from jax.experimental import pallas as pl
from jax.experimental.pallas import tpu as pltpu
from jax.experimental.pallas import tpu_sc as plsc

R = 512
C = 32
NC = 2
NS = 16
NW = NC * NS
BC = 128
SCH = 8
SBC = BC * SCH
LANES = 16


def _prep_body(xyz_ref, idx_ref, w_ref):
    xyz_b = xyz_ref[...].astype(jnp.bfloat16).astype(jnp.float32)
    x = xyz_b[0:1, :]
    y = xyz_b[1:2, :]
    z = xyz_b[2:3, :]
    idx_rows = []
    w_rows = []
    for p, (gx, gy) in enumerate(((y, x), (z, x), (y, z))):
        ixf = (gx + 1.0) * (0.5 * (R - 1))
        iyf = (gy + 1.0) * (0.5 * (R - 1))
        ix0 = jnp.clip(jnp.floor(ixf), 0.0, R - 2.0)
        iy0 = jnp.clip(jnp.floor(iyf), 0.0, R - 2.0)
        fx = ixf - ix0
        fy = iyf - iy0
        base = (p * R * R) + iy0.astype(jnp.int32) * R + ix0.astype(jnp.int32)
        for dy in (0, 1):
            for dx in (0, 1):
                idx_rows.append(base + (dy * R + dx))
                w_rows.append((fx if dx else 1.0 - fx) * (fy if dy else 1.0 - fy))
    idx_ref[...] = jnp.concatenate(idx_rows, axis=0)
    w_ref[...] = jnp.concatenate(w_rows, axis=0)


@functools.lru_cache(maxsize=None)
def _make_prep(m):
    bp = 8192
    grid = (m // bp,)
    return pl.pallas_call(
        _prep_body,
        grid=grid,
        in_specs=[pl.BlockSpec((3, bp), lambda i: (0, i))],
        out_specs=[
            pl.BlockSpec((12, bp), lambda i: (0, i)),
            pl.BlockSpec((12, bp), lambda i: (0, i)),
        ],
        out_shape=[
            jax.ShapeDtypeStruct((12, m), jnp.int32),
            jax.ShapeDtypeStruct((12, m), jnp.float32),
        ],
    )


@functools.lru_cache(maxsize=None)
def _make_sc(m):
    ppt = m // NW
    nsch = ppt // SBC
    mesh = plsc.VectorSubcoreMesh(core_axis_name="c", subcore_axis_name="s")

    def compute_chunk(cc, rows, w_v, out_v, lanes):
        hi_mask = jnp.full((LANES,), -65536, jnp.int32)

        def group_body(g, carry2):
            rowbase = g * LANES
            row_lanes = lanes + rowbase
            wvs = [plsc.load_gather(w_v, [row_lanes + (j * SBC + cc * BC)])
                   for j in range(12)]
            out_lanes = row_lanes + cc * BC
            for w in range(C // 2):
                widx = jnp.full((LANES,), w, jnp.int32)
                acc_lo = jnp.zeros((LANES,), jnp.float32)
                acc_hi = jnp.zeros((LANES,), jnp.float32)
                for j in range(12):
                    word = plsc.load_gather(rows[j], [row_lanes, widx])
                    lo = plsc.bitcast(word << 16, jnp.float32)
                    hi = plsc.bitcast(word & hi_mask, jnp.float32)
                    acc_lo = acc_lo + wvs[j] * lo
                    acc_hi = acc_hi + wvs[j] * hi
                plsc.store_scatter(
                    out_v, [out_lanes, jnp.full((LANES,), 2 * w, jnp.int32)],
                    acc_lo)
                plsc.store_scatter(
                    out_v, [out_lanes, jnp.full((LANES,), 2 * w + 1, jnp.int32)],
                    acc_hi)
            return carry2

        lax.fori_loop(0, BC // LANES, group_body, 0)

    def body(table_hbm, idx_hbm, w_hbm, out_hbm,
             idx_v, w_v, rows_a, rows_b, out_v, sem_s, sem_a, sem_b):
        wid = lax.axis_index("s") * NC + lax.axis_index("c")
        lanes = lax.iota(jnp.int32, LANES)
        rows = (rows_a, rows_b)
        sems = (sem_a, sem_b)

        def fire_gathers(cc):
            par = cc % 2
            return [
                pltpu.async_copy(
                    table_hbm.at[idx_v.at[j, pl.ds(cc * BC, BC)]],
                    rows[par][j], sems[par])
                for j in range(12)
            ]

        def sch_body(s, carry):
            base = wid * ppt + s * SBC
            stages = []
            for j in range(12):
                stages.append(pltpu.async_copy(
                    idx_hbm.at[j, pl.ds(base, SBC)], idx_v.at[j], sem_s))
                stages.append(pltpu.async_copy(
                    w_hbm.at[j, pl.ds(base, SBC)],
                    w_v.at[pl.ds(j * SBC, SBC)], sem_s))
            for st in stages:
                st.wait()
            pending = fire_gathers(0)
            for cc in range(SCH):
                nxt = fire_gathers(cc + 1) if cc + 1 < SCH else []
                for g_ in pending:
                    g_.wait()
                compute_chunk(cc, rows[cc % 2], w_v, out_v, lanes)
                pending = nxt
            pltpu.sync_copy(out_v, out_hbm.at[pl.ds(base, SBC)])
            return carry

        lax.fori_loop(0, nsch, sch_body, 0)

    return pl.kernel(
        body,
        out_type=jax.ShapeDtypeStruct((m, C), jnp.float32),
        mesh=mesh,
        compiler_params=pltpu.CompilerParams(
            needs_layout_passes=False, use_tc_tiling_on_sc=False),
        scratch_types=[
            pltpu.VMEM((12, SBC), jnp.int32),
            pltpu.VMEM((12 * SBC,), jnp.float32),
            [pltpu.VMEM((BC, C // 2), jnp.int32) for _ in range(12)],
            [pltpu.VMEM((BC, C // 2), jnp.int32) for _ in range(12)],
            pltpu.VMEM((SBC, C), jnp.float32),
            pltpu.SemaphoreType.DMA,
            pltpu.SemaphoreType.DMA,
            pltpu.SemaphoreType.DMA,
        ],
    )


def kernel(xyz, oid, triplane):
    m = xyz.shape[0]
    if triplane.shape[0] == 1:
        planes = triplane.reshape(triplane.shape[1:])
    else:
        planes = triplane[oid]
    t32 = jnp.transpose(planes, (0, 2, 3, 1))
    bits = jax.lax.bitcast_convert_type(t32, jnp.int32)
    h = jax.lax.shift_right_logical(
        bits + 0x7FFF + (jax.lax.shift_right_logical(bits, 16) & 1), 16)
    words = (h[..., 0::2] & 0xFFFF) | (h[..., 1::2] << 16)
    table = words.reshape(3 * R * R, C // 2)
    xyz_t = xyz.T
    idx6, w12 = _make_prep(m)(xyz_t)
    return _make_sc(m)(table, idx6, w12)
